# baseline (speedup 1.0000x reference)
import functools

import jax
import jax.numpy as jnp
from jax import lax
from jax.experimental import pallas as pl
from jax.experimental.pallas import tpu as pltpu
from jax.experimental.pallas import tpu_sc as plsc

try:
    _info = plsc.get_sparse_core_info()
    _NC, _NS, _L = _info.num_cores, _info.num_subcores, _info.num_lanes
except Exception:
    _NC, _NS, _L = 2, 16, 16

_NW = _NC * _NS
_BATCH = 16384
_DIM = 128
_BPW = _BATCH // _NW
_C = 64
_NCHUNK = _BPW // _C
_RU = 4


def _unpk(words):
    ab = plsc.bitcast(words, jnp.bfloat16)
    return plsc.unpack(ab, format=plsc.PackFormat.INTERLEAVED,
                       preferred_element_type=jnp.float32)


def _make_kernel():
    mesh = plsc.VectorSubcoreMesh(core_axis_name="c", subcore_axis_name="s")

    @functools.partial(
        pl.kernel,
        mesh=mesh,
        compiler_params=pltpu.CompilerParams(needs_layout_passes=False,
                                             use_tc_tiling_on_sc=False),
        out_type=(
            jax.ShapeDtypeStruct((_BATCH,), jnp.float32),
            jax.ShapeDtypeStruct((_BATCH,), jnp.float32),
        ),
        scratch_types=[
            pltpu.VMEM((_BPW,), jnp.int32),
            pltpu.VMEM((_BPW,), jnp.int32),
            pltpu.VMEM((_BPW,), jnp.int32),
            pltpu.VMEM((_BPW,), jnp.int32),
            pltpu.VMEM((_C, _DIM // 2), jnp.int32),
            pltpu.VMEM((_C, _DIM // 2), jnp.int32),
            pltpu.VMEM((_C, _DIM // 2), jnp.int32),
            pltpu.VMEM((_C, _DIM // 2), jnp.int32),
            pltpu.VMEM((_C, _DIM // 2), jnp.int32),
            pltpu.VMEM((_C, _DIM // 2), jnp.int32),
            pltpu.VMEM((_C, _DIM // 2), jnp.int32),
            pltpu.VMEM((_C, _DIM // 2), jnp.int32),
            pltpu.VMEM((_BPW,), jnp.float32),
            pltpu.VMEM((_BPW,), jnp.float32),
            pltpu.SemaphoreType.DMA,
            pltpu.SemaphoreType.DMA,
        ],
    )
    def transe_kernel(entity_hbm, relation_hbm, heads_hbm, rels_hbm,
                      tails_hbm, negs_hbm, pos_out, neg_out,
                      hidx, ridx, tidx, nidx,
                      hb0, rb0, tb0, nb0, hb1, rb1, tb1, nb1,
                      pos_buf, neg_buf, sem0, sem1):
        wid = lax.axis_index("s") * _NC + lax.axis_index("c")
        base = wid * _BPW
        lanes = lax.iota(jnp.int32, _L)

        pltpu.sync_copy(heads_hbm.at[pl.ds(base, _BPW)], hidx)
        pltpu.sync_copy(rels_hbm.at[pl.ds(base, _BPW)], ridx)
        pltpu.sync_copy(tails_hbm.at[pl.ds(base, _BPW)], tidx)
        pltpu.sync_copy(negs_hbm.at[pl.ds(base, _BPW)], nidx)

        bufs = ((hb0, rb0, tb0, nb0, sem0), (hb1, rb1, tb1, nb1, sem1))

        def issue(c):
            hb, rb, tb, nb, sem = bufs[c % 2]
            cb = c * _C
            return (
                pltpu.async_copy(entity_hbm.at[hidx.at[pl.ds(cb, _C)]],
                                 hb, sem),
                pltpu.async_copy(relation_hbm.at[ridx.at[pl.ds(cb, _C)]],
                                 rb, sem),
                pltpu.async_copy(entity_hbm.at[tidx.at[pl.ds(cb, _C)]],
                                 tb, sem),
                pltpu.async_copy(entity_hbm.at[nidx.at[pl.ds(cb, _C)]],
                                 nb, sem),
            )

        xors = [jnp.bitwise_xor(lanes, x) for x in (8, 4, 2, 1)]
        sel = [lanes == k for k in range(_RU)]
        mask4 = lanes < _RU

        dnums = lax.GatherDimensionNumbers(
            offset_dims=(), collapsed_slice_dims=(0,), start_index_map=(0,))

        def perm(v, x):
            return lax.gather(v, x[:, None], dnums, (1,),
                              mode=lax.GatherScatterMode.PROMISE_IN_BOUNDS)

        def lane_sum(v):
            for x in xors:
                v = v + perm(v, x)
            return v

        pending = issue(0)
        for c in range(_NCHUNK):
            nxt = issue(c + 1) if c + 1 < _NCHUNK else None
            for cp in pending:
                cp.wait()
            hb, rb, tb, nb, _ = bufs[c % 2]
            cb = c * _C

            def body(q, carry):
                i0 = q * _RU
                rp = jnp.zeros((_L,), jnp.float32)
                rn = jnp.zeros((_L,), jnp.float32)
                for k in range(_RU):
                    i = i0 + k
                    accp = jnp.zeros((_L,), jnp.float32)
                    accn = jnp.zeros((_L,), jnp.float32)
                    for j in range(_DIM // (2 * _L)):
                        sl = pl.ds(j * _L, _L)
                        h0, h1 = _unpk(hb[i, sl])
                        r0, r1 = _unpk(rb[i, sl])
                        t0, t1 = _unpk(tb[i, sl])
                        n0, n1 = _unpk(nb[i, sl])
                        hr0 = h0 + r0
                        hr1 = h1 + r1
                        accp = accp + jnp.abs(hr0 - t0) + jnp.abs(hr1 - t1)
                        accn = accn + jnp.abs(hr0 - n0) + jnp.abs(hr1 - n1)
                    rp = jnp.where(sel[k], lane_sum(accp), rp)
                    rn = jnp.where(sel[k], lane_sum(accn), rn)
                out_idx = jnp.full((_L,), cb + i0, jnp.int32) + lanes
                plsc.store_scatter(pos_buf, [out_idx], rp, mask=mask4)
                plsc.store_scatter(neg_buf, [out_idx], rn, mask=mask4)
                return carry

            lax.fori_loop(0, _C // _RU, body, 0)
            pending = nxt

        pltpu.sync_copy(pos_buf, pos_out.at[pl.ds(base, _BPW)])
        pltpu.sync_copy(neg_buf, neg_out.at[pl.ds(base, _BPW)])

    return transe_kernel


_transe = _make_kernel()


def kernel(entity_emb, relation_emb, heads, relations, tails, negative_tails):
    entity_emb = lax.bitcast_convert_type(
        entity_emb.astype(jnp.bfloat16).reshape(-1, _DIM // 2, 2), jnp.int32)
    relation_emb = lax.bitcast_convert_type(
        relation_emb.astype(jnp.bfloat16).reshape(-1, _DIM // 2, 2), jnp.int32)
    heads = heads.astype(jnp.int32)
    relations = relations.astype(jnp.int32)
    tails = tails.astype(jnp.int32)
    negative_tails = negative_tails.astype(jnp.int32)
    pos, neg = _transe(entity_emb, relation_emb, heads, relations,
                       tails, negative_tails)
    return (pos, neg)


# speedup vs baseline: 11.4130x; 11.4130x over previous
import functools

import jax
import jax.numpy as jnp
from jax import lax
from jax.experimental import pallas as pl
from jax.experimental.pallas import tpu as pltpu
from jax.experimental.pallas import tpu_sc as plsc

try:
    _info = plsc.get_sparse_core_info()
    _NC, _NS, _L = _info.num_cores, _info.num_subcores, _info.num_lanes
except Exception:
    _NC, _NS, _L = 2, 16, 16

_NW = _NC * _NS
_BATCH = 16384
_DIM = 128
_BPW = _BATCH // _NW
_C = 64
_NCHUNK = _BPW // _C


def _make_kernel():
    mesh = plsc.VectorSubcoreMesh(core_axis_name="c", subcore_axis_name="s")

    @functools.partial(
        pl.kernel,
        mesh=mesh,
        compiler_params=pltpu.CompilerParams(needs_layout_passes=False),
        out_type=(
            jax.ShapeDtypeStruct((_BATCH,), jnp.float32),
            jax.ShapeDtypeStruct((_BATCH,), jnp.float32),
        ),
        scratch_types=[
            pltpu.VMEM((_BPW,), jnp.int32),
            pltpu.VMEM((_BPW,), jnp.int32),
            pltpu.VMEM((_BPW,), jnp.int32),
            pltpu.VMEM((_BPW,), jnp.int32),
            pltpu.VMEM((_C, _DIM), jnp.float32),
            pltpu.VMEM((_C, _DIM), jnp.float32),
            pltpu.VMEM((_C, _DIM), jnp.float32),
            pltpu.VMEM((_C, _DIM), jnp.float32),
            pltpu.VMEM((_C, _DIM), jnp.float32),
            pltpu.VMEM((_C, _DIM), jnp.float32),
            pltpu.VMEM((_C, _DIM), jnp.float32),
            pltpu.VMEM((_C, _DIM), jnp.float32),
            pltpu.VMEM((_C, _DIM), jnp.float32),
            pltpu.VMEM((_C, _DIM), jnp.float32),
            pltpu.VMEM((_C, _DIM), jnp.float32),
            pltpu.VMEM((_C, _DIM), jnp.float32),
            pltpu.VMEM((_BPW,), jnp.float32),
            pltpu.VMEM((_BPW,), jnp.float32),
            pltpu.SemaphoreType.DMA,
            pltpu.SemaphoreType.DMA,
            pltpu.SemaphoreType.DMA,
        ],
    )
    def transe_kernel(entity_hbm, relation_hbm, heads_hbm, rels_hbm,
                      tails_hbm, negs_hbm, pos_out, neg_out,
                      hidx, ridx, tidx, nidx,
                      hb0, rb0, tb0, nb0, hb1, rb1, tb1, nb1,
                      hb2, rb2, tb2, nb2,
                      pos_buf, neg_buf, sem0, sem1, sem2):
        wid = lax.axis_index("s") * _NC + lax.axis_index("c")
        base = wid * _BPW
        lanes = lax.iota(jnp.int32, _L)

        pltpu.sync_copy(heads_hbm.at[pl.ds(base, _BPW)], hidx)
        pltpu.sync_copy(rels_hbm.at[pl.ds(base, _BPW)], ridx)
        pltpu.sync_copy(tails_hbm.at[pl.ds(base, _BPW)], tidx)
        pltpu.sync_copy(negs_hbm.at[pl.ds(base, _BPW)], nidx)

        bufs = ((hb0, rb0, tb0, nb0, sem0), (hb1, rb1, tb1, nb1, sem1),
                (hb2, rb2, tb2, nb2, sem2))

        def issue(c):
            hb, rb, tb, nb, sem = bufs[c % 3]
            cb = c * _C
            return (
                pltpu.async_copy(entity_hbm.at[hidx.at[pl.ds(cb, _C)]],
                                 hb, sem),
                pltpu.async_copy(relation_hbm.at[ridx.at[pl.ds(cb, _C)]],
                                 rb, sem),
                pltpu.async_copy(entity_hbm.at[tidx.at[pl.ds(cb, _C)]],
                                 tb, sem),
                pltpu.async_copy(entity_hbm.at[nidx.at[pl.ds(cb, _C)]],
                                 nb, sem),
            )

        last = lanes == (_L - 1)

        queue = [issue(0), issue(1)]
        for c in range(_NCHUNK):
            if c + 2 < _NCHUNK:
                queue.append(issue(c + 2))
            for cp in queue.pop(0):
                cp.wait()
            hb, rb, tb, nb, _ = bufs[c % 3]
            cb = c * _C

            def body(i, carry):
                accp = jnp.zeros((_L,), jnp.float32)
                accn = jnp.zeros((_L,), jnp.float32)
                for j in range(_DIM // _L):
                    sl = pl.ds(j * _L, _L)
                    hr = hb[i, sl] + rb[i, sl]
                    accp = accp + jnp.abs(hr - tb[i, sl])
                    accn = accn + jnp.abs(hr - nb[i, sl])
                out_idx = jnp.full((_L,), cb + i, jnp.int32)
                plsc.store_scatter(pos_buf, [out_idx], plsc.cumsum(accp),
                                   mask=last)
                plsc.store_scatter(neg_buf, [out_idx], plsc.cumsum(accn),
                                   mask=last)
                return carry

            lax.fori_loop(0, _C, body, 0)

        pltpu.sync_copy(pos_buf, pos_out.at[pl.ds(base, _BPW)])
        pltpu.sync_copy(neg_buf, neg_out.at[pl.ds(base, _BPW)])

    return transe_kernel


_transe = _make_kernel()


def kernel(entity_emb, relation_emb, heads, relations, tails, negative_tails):
    heads = heads.astype(jnp.int32)
    relations = relations.astype(jnp.int32)
    tails = tails.astype(jnp.int32)
    negative_tails = negative_tails.astype(jnp.int32)
    pos, neg = _transe(entity_emb, relation_emb, heads, relations,
                       tails, negative_tails)
    return (pos, neg)


# speedup vs baseline: 11.4822x; 1.0061x over previous
import functools

import jax
import jax.numpy as jnp
from jax import lax
from jax.experimental import pallas as pl
from jax.experimental.pallas import tpu as pltpu
from jax.experimental.pallas import tpu_sc as plsc

try:
    _info = plsc.get_sparse_core_info()
    _NC, _NS, _L = _info.num_cores, _info.num_subcores, _info.num_lanes
except Exception:
    _NC, _NS, _L = 2, 16, 16

_NW = _NC * _NS
_BATCH = 16384
_DIM = 128
_BPW = _BATCH // _NW
_C = 64
_NCHUNK = _BPW // _C


def _make_kernel():
    mesh = plsc.VectorSubcoreMesh(core_axis_name="c", subcore_axis_name="s")

    @functools.partial(
        pl.kernel,
        mesh=mesh,
        compiler_params=pltpu.CompilerParams(needs_layout_passes=False),
        out_type=(
            jax.ShapeDtypeStruct((_BATCH,), jnp.float32),
            jax.ShapeDtypeStruct((_BATCH,), jnp.float32),
        ),
        scratch_types=[
            pltpu.VMEM((_BPW,), jnp.int32),
            pltpu.VMEM((_BPW,), jnp.int32),
            pltpu.VMEM((_BPW,), jnp.int32),
            pltpu.VMEM((_BPW,), jnp.int32),
            pltpu.VMEM((_C, _DIM), jnp.float32),
            pltpu.VMEM((_C, _DIM), jnp.float32),
            pltpu.VMEM((_C, _DIM), jnp.float32),
            pltpu.VMEM((_C, _DIM), jnp.float32),
            pltpu.VMEM((_C, _DIM), jnp.float32),
            pltpu.VMEM((_C, _DIM), jnp.float32),
            pltpu.VMEM((_C, _DIM), jnp.float32),
            pltpu.VMEM((_C, _DIM), jnp.float32),
            pltpu.VMEM((_BPW,), jnp.float32),
            pltpu.VMEM((_BPW,), jnp.float32),
            pltpu.SemaphoreType.DMA,
            pltpu.SemaphoreType.DMA,
        ],
    )
    def transe_kernel(entity_hbm, relation_hbm, heads_hbm, rels_hbm,
                      tails_hbm, negs_hbm, pos_out, neg_out,
                      hidx, ridx, tidx, nidx,
                      hb0, rb0, tb0, nb0, hb1, rb1, tb1, nb1,
                      pos_buf, neg_buf, sem0, sem1):
        wid = lax.axis_index("s") * _NC + lax.axis_index("c")
        base = wid * _BPW
        lanes = lax.iota(jnp.int32, _L)

        pltpu.sync_copy(heads_hbm.at[pl.ds(base, _BPW)], hidx)
        pltpu.sync_copy(rels_hbm.at[pl.ds(base, _BPW)], ridx)
        pltpu.sync_copy(tails_hbm.at[pl.ds(base, _BPW)], tidx)
        pltpu.sync_copy(negs_hbm.at[pl.ds(base, _BPW)], nidx)

        bufs = ((hb0, rb0, tb0, nb0, sem0), (hb1, rb1, tb1, nb1, sem1))

        def issue(c):
            hb, rb, tb, nb, sem = bufs[c % 2]
            cb = c * _C
            return (
                pltpu.async_copy(entity_hbm.at[hidx.at[pl.ds(cb, _C)]],
                                 hb, sem),
                pltpu.async_copy(relation_hbm.at[ridx.at[pl.ds(cb, _C)]],
                                 rb, sem),
                pltpu.async_copy(entity_hbm.at[tidx.at[pl.ds(cb, _C)]],
                                 tb, sem),
                pltpu.async_copy(entity_hbm.at[nidx.at[pl.ds(cb, _C)]],
                                 nb, sem),
            )

        first = lanes == 0
        xors = [jnp.bitwise_xor(lanes, x) for x in (8, 4, 2, 1)]
        dnums = lax.GatherDimensionNumbers(
            offset_dims=(), collapsed_slice_dims=(0,), start_index_map=(0,))

        def lane_sum(v):
            for x in xors:
                v = v + lax.gather(
                    v, x[:, None], dnums, (1,),
                    mode=lax.GatherScatterMode.PROMISE_IN_BOUNDS)
            return v

        pending = issue(0)
        for c in range(_NCHUNK):
            nxt = issue(c + 1) if c + 1 < _NCHUNK else None
            for cp in pending:
                cp.wait()
            hb, rb, tb, nb, _ = bufs[c % 2]
            cb = c * _C

            def body(i, carry):
                accp = jnp.zeros((_L,), jnp.float32)
                accn = jnp.zeros((_L,), jnp.float32)
                for j in range(_DIM // _L):
                    sl = pl.ds(j * _L, _L)
                    hr = hb[i, sl] + rb[i, sl]
                    accp = accp + jnp.abs(hr - tb[i, sl])
                    accn = accn + jnp.abs(hr - nb[i, sl])
                out_idx = jnp.full((_L,), cb + i, jnp.int32)
                plsc.store_scatter(pos_buf, [out_idx], lane_sum(accp),
                                   mask=first)
                plsc.store_scatter(neg_buf, [out_idx], lane_sum(accn),
                                   mask=first)
                return carry

            lax.fori_loop(0, _C, body, 0)
            pending = nxt

        pltpu.sync_copy(pos_buf, pos_out.at[pl.ds(base, _BPW)])
        pltpu.sync_copy(neg_buf, neg_out.at[pl.ds(base, _BPW)])

    return transe_kernel


_transe = _make_kernel()


def kernel(entity_emb, relation_emb, heads, relations, tails, negative_tails):
    heads = heads.astype(jnp.int32)
    relations = relations.astype(jnp.int32)
    tails = tails.astype(jnp.int32)
    negative_tails = negative_tails.astype(jnp.int32)
    pos, neg = _transe(entity_emb, relation_emb, heads, relations,
                       tails, negative_tails)
    return (pos, neg)


# speedup vs baseline: 11.7220x; 1.0209x over previous
import functools

import jax
import jax.numpy as jnp
from jax import lax
from jax.experimental import pallas as pl
from jax.experimental.pallas import tpu as pltpu
from jax.experimental.pallas import tpu_sc as plsc

try:
    _info = plsc.get_sparse_core_info()
    _NC, _NS, _L = _info.num_cores, _info.num_subcores, _info.num_lanes
except Exception:
    _NC, _NS, _L = 2, 16, 16

_NW = _NC * _NS
_BATCH = 16384
_DIM = 128
_BPW = _BATCH // _NW
_C = 64
_NCHUNK = _BPW // _C


def _make_kernel():
    mesh = plsc.VectorSubcoreMesh(core_axis_name="c", subcore_axis_name="s")

    @functools.partial(
        pl.kernel,
        mesh=mesh,
        compiler_params=pltpu.CompilerParams(needs_layout_passes=False),
        out_type=(
            jax.ShapeDtypeStruct((_BATCH,), jnp.float32),
            jax.ShapeDtypeStruct((_BATCH,), jnp.float32),
        ),
        scratch_types=[
            pltpu.VMEM((_BPW,), jnp.int32),
            pltpu.VMEM((_BPW,), jnp.int32),
            pltpu.VMEM((_BPW,), jnp.int32),
            pltpu.VMEM((_BPW,), jnp.int32),
            pltpu.VMEM((_C, _DIM), jnp.float32),
            pltpu.VMEM((_C, _DIM), jnp.float32),
            pltpu.VMEM((_C, _DIM), jnp.float32),
            pltpu.VMEM((_C, _DIM), jnp.float32),
            pltpu.VMEM((_C, _DIM), jnp.float32),
            pltpu.VMEM((_C, _DIM), jnp.float32),
            pltpu.VMEM((_C, _DIM), jnp.float32),
            pltpu.VMEM((_C, _DIM), jnp.float32),
            pltpu.VMEM((_BPW,), jnp.float32),
            pltpu.VMEM((_BPW,), jnp.float32),
            pltpu.SemaphoreType.DMA,
            pltpu.SemaphoreType.DMA,
        ],
    )
    def transe_kernel(entity_hbm, relation_hbm, heads_hbm, rels_hbm,
                      tails_hbm, negs_hbm, pos_out, neg_out,
                      hidx, ridx, tidx, nidx,
                      hb0, rb0, tb0, nb0, hb1, rb1, tb1, nb1,
                      pos_buf, neg_buf, sem0, sem1):
        wid = lax.axis_index("s") * _NC + lax.axis_index("c")
        base = wid * _BPW
        lanes = lax.iota(jnp.int32, _L)

        pltpu.sync_copy(heads_hbm.at[pl.ds(base, _BPW)], hidx)
        pltpu.sync_copy(rels_hbm.at[pl.ds(base, _BPW)], ridx)
        pltpu.sync_copy(tails_hbm.at[pl.ds(base, _BPW)], tidx)
        pltpu.sync_copy(negs_hbm.at[pl.ds(base, _BPW)], nidx)

        bufs = ((hb0, rb0, tb0, nb0, sem0), (hb1, rb1, tb1, nb1, sem1))

        def issue(c):
            hb, rb, tb, nb, sem = bufs[c % 2]
            cb = c * _C
            return (
                pltpu.async_copy(entity_hbm.at[hidx.at[pl.ds(cb, _C)]],
                                 hb, sem),
                pltpu.async_copy(relation_hbm.at[ridx.at[pl.ds(cb, _C)]],
                                 rb, sem),
                pltpu.async_copy(entity_hbm.at[tidx.at[pl.ds(cb, _C)]],
                                 tb, sem),
                pltpu.async_copy(entity_hbm.at[nidx.at[pl.ds(cb, _C)]],
                                 nb, sem),
            )

        last = lanes == (_L - 1)

        pending = issue(0)
        for c in range(_NCHUNK):
            nxt = issue(c + 1) if c + 1 < _NCHUNK else None
            for cp in pending:
                cp.wait()
            hb, rb, tb, nb, _ = bufs[c % 2]
            cb = c * _C

            def body(i, carry):
                accp = jnp.zeros((_L,), jnp.float32)
                accn = jnp.zeros((_L,), jnp.float32)
                for j in range(_DIM // _L):
                    sl = pl.ds(j * _L, _L)
                    hr = hb[i, sl] + rb[i, sl]
                    accp = accp + jnp.abs(hr - tb[i, sl])
                    accn = accn + jnp.abs(hr - nb[i, sl])
                out_idx = jnp.full((_L,), cb + i, jnp.int32)
                plsc.store_scatter(pos_buf, [out_idx], plsc.cumsum(accp),
                                   mask=last)
                plsc.store_scatter(neg_buf, [out_idx], plsc.cumsum(accn),
                                   mask=last)
                return carry

            lax.fori_loop(0, _C, body, 0)
            pending = nxt

        pltpu.sync_copy(pos_buf, pos_out.at[pl.ds(base, _BPW)])
        pltpu.sync_copy(neg_buf, neg_out.at[pl.ds(base, _BPW)])

    return transe_kernel


_transe = _make_kernel()


def kernel(entity_emb, relation_emb, heads, relations, tails, negative_tails):
    heads = heads.astype(jnp.int32)
    relations = relations.astype(jnp.int32)
    tails = tails.astype(jnp.int32)
    negative_tails = negative_tails.astype(jnp.int32)
    pos, neg = _transe(entity_emb, relation_emb, heads, relations,
                       tails, negative_tails)
    return (pos, neg)
